# baseline (speedup 1.0000x reference)
import jax
import jax.numpy as jnp
from jax.experimental import pallas as pl
from jax.experimental.pallas import tpu as pltpu
from jax.experimental.pallas import tpu_sc as plsc

EMB = 128
HID = 512
B = 1024
T = 200

_GATHER_WINDOW = 128
_N_IDX = B * T


def _sc_gather(table, idx):
    n = _N_IDX
    e = table.shape[1]
    mesh = plsc.VectorSubcoreMesh(core_axis_name="core", subcore_axis_name="subcore")

    @pl.kernel(out_type=jax.ShapeDtypeStruct((n, e), table.dtype), mesh=mesh)
    def gather_kernel(tab_hbm, i_hbm, o_hbm):
        def body(i_vmem, o_vmem):
            pltpu.sync_copy(tab_hbm.at[i_vmem.at[0]], o_vmem)

        pltpu.emit_pipeline(
            body,
            grid=(n // _GATHER_WINDOW,),
            in_specs=[pl.BlockSpec((1, _GATHER_WINDOW), index_map=lambda i: (0, i))],
            out_specs=[pl.BlockSpec((_GATHER_WINDOW, e), index_map=lambda i: (i, 0))],
            core_axis_name=("core", "subcore"),
            dimension_semantics=(pltpu.PARALLEL,),
        )(i_hbm, o_hbm)

    return gather_kernel(table, idx)


def _gates(g, first, c_ref):
    half = jnp.bfloat16(0.5)
    i = half + half * jnp.tanh(g[:, :HID])
    f = half + half * jnp.tanh(g[:, HID:2 * HID])
    gg = jnp.tanh(g[:, 2 * HID:3 * HID])
    o = half + half * jnp.tanh(g[:, 3 * HID:])
    c_prev = jnp.where(first, jnp.float32(0), c_ref[...])
    c2 = f.astype(jnp.float32) * c_prev + (i * gg).astype(jnp.float32)
    c_ref[...] = c2
    return o * jnp.tanh(c2).astype(jnp.bfloat16)


def _full_spec(a):
    nd = a.ndim
    return pl.BlockSpec(a.shape, lambda t, _n=nd: (0,) * _n)


def _l0_body(xf_ref, xb_ref, wf_ref, bf_ref, wb_ref, bb_ref,
             hfo_ref, hbo_ref, xcf, cf, xcb, cb):
    t = pl.program_id(0)
    first = t == 0

    def step(x_ref, w_ref, b_ref, xc, c, out_ref, j, fst):
        if fst is not False:
            @pl.when(fst)
            def _():
                xc[:, EMB:] = jnp.zeros_like(xc[:, EMB:])

        xc[:, :EMB] = x_ref[j].astype(jnp.bfloat16)
        g = (jnp.dot(xc[...], w_ref[...], preferred_element_type=jnp.float32)
             ).astype(jnp.bfloat16) + b_ref[...]
        hb = _gates(g, fst, c)
        xc[:, EMB:] = hb
        out_ref[j] = hb

    step(xf_ref, wf_ref, bf_ref, xcf, cf, hfo_ref, 0, first)
    step(xb_ref, wb_ref, bb_ref, xcb, cb, hbo_ref, 3, first)
    step(xf_ref, wf_ref, bf_ref, xcf, cf, hfo_ref, 1, False)
    step(xb_ref, wb_ref, bb_ref, xcb, cb, hbo_ref, 2, False)
    step(xf_ref, wf_ref, bf_ref, xcf, cf, hfo_ref, 2, False)
    step(xb_ref, wb_ref, bb_ref, xcb, cb, hbo_ref, 1, False)
    step(xf_ref, wf_ref, bf_ref, xcf, cf, hfo_ref, 3, False)
    step(xb_ref, wb_ref, bb_ref, xcb, cb, hbo_ref, 0, False)


def _bilstm_layer0(emb, wcf, bf, wcb, bb):
    out_shape = [jax.ShapeDtypeStruct((T, B, HID), jnp.bfloat16),
                 jax.ShapeDtypeStruct((T, B, HID), jnp.bfloat16)]
    k0 = EMB + HID
    return pl.pallas_call(
        _l0_body,
        grid=(T // 4,),
        in_specs=[
            pl.BlockSpec((4, B, EMB), lambda t: (t, 0, 0)),
            pl.BlockSpec((4, B, EMB), lambda t: (T // 4 - 1 - t, 0, 0)),
            _full_spec(wcf), _full_spec(bf), _full_spec(wcb), _full_spec(bb),
        ],
        out_specs=[
            pl.BlockSpec((4, B, HID), lambda t: (t, 0, 0)),
            pl.BlockSpec((4, B, HID), lambda t: (T // 4 - 1 - t, 0, 0)),
        ],
        out_shape=out_shape,
        scratch_shapes=[pltpu.VMEM((B, k0), jnp.bfloat16),
                        pltpu.VMEM((B, HID), jnp.float32),
                        pltpu.VMEM((B, k0), jnp.bfloat16),
                        pltpu.VMEM((B, HID), jnp.float32)],
        compiler_params=pltpu.CompilerParams(dimension_semantics=("arbitrary",)),
    )(emb, emb, wcf, bf, wcb, bb)


def _l1_body(hff_ref, hbf_ref, hfb_ref, hbb_ref,
             wf_ref, bf_ref, wb_ref, bb_ref,
             fwf_ref, fwb_ref, fcb_ref,
             out_ref, xcf, cf, xcb, cb):
    t = pl.program_id(0)
    first = t == 0

    def step(in1_ref, in2_ref, w_ref, b_ref, xc, c, j, fst):
        if fst is not False:
            @pl.when(fst)
            def _():
                xc[:, 2 * HID:] = jnp.zeros_like(xc[:, 2 * HID:])

        xc[:, :HID] = in1_ref[j]
        xc[:, HID:2 * HID] = in2_ref[j]
        g = (jnp.dot(xc[...], w_ref[...], preferred_element_type=jnp.float32)
             ).astype(jnp.bfloat16) + b_ref[...]
        hb = _gates(g, fst, c)
        xc[:, 2 * HID:] = hb

    step(hff_ref, hbf_ref, wf_ref, bf_ref, xcf, cf, 0, first)
    step(hfb_ref, hbb_ref, wb_ref, bb_ref, xcb, cb, 3, first)
    step(hff_ref, hbf_ref, wf_ref, bf_ref, xcf, cf, 1, False)
    step(hfb_ref, hbb_ref, wb_ref, bb_ref, xcb, cb, 2, False)
    step(hff_ref, hbf_ref, wf_ref, bf_ref, xcf, cf, 2, False)
    step(hfb_ref, hbb_ref, wb_ref, bb_ref, xcb, cb, 1, False)
    step(hff_ref, hbf_ref, wf_ref, bf_ref, xcf, cf, 3, False)
    step(hfb_ref, hbb_ref, wb_ref, bb_ref, xcb, cb, 0, False)

    @pl.when(t == T // 4 - 1)
    def _():
        vf = jnp.sum(xcf[:, 2 * HID:].astype(jnp.float32) * fwf_ref[...],
                     axis=1, keepdims=True)
        vb = jnp.sum(xcb[:, 2 * HID:].astype(jnp.float32) * fwb_ref[...],
                     axis=1, keepdims=True)
        z = vf + vb + fcb_ref[...]
        out_ref[...] = 0.5 + 0.5 * jnp.tanh(0.5 * z)


def _bilstm_layer1_fc(hf0, hb0, wcf, bf, wcb, bb, fwf, fwb, fcb):
    seq_spec_f = pl.BlockSpec((4, B, HID), lambda t: (t, 0, 0))
    seq_spec_b = pl.BlockSpec((4, B, HID), lambda t: (T // 4 - 1 - t, 0, 0))
    k1 = 3 * HID
    return pl.pallas_call(
        _l1_body,
        grid=(T // 4,),
        in_specs=[
            seq_spec_f, seq_spec_f, seq_spec_b, seq_spec_b,
            _full_spec(wcf), _full_spec(bf), _full_spec(wcb), _full_spec(bb),
            _full_spec(fwf), _full_spec(fwb), _full_spec(fcb),
        ],
        out_specs=pl.BlockSpec((B, 1), lambda t: (0, 0)),
        out_shape=jax.ShapeDtypeStruct((B, 1), jnp.float32),
        scratch_shapes=[pltpu.VMEM((B, k1), jnp.bfloat16),
                        pltpu.VMEM((B, HID), jnp.float32),
                        pltpu.VMEM((B, k1), jnp.bfloat16),
                        pltpu.VMEM((B, HID), jnp.float32)],
        compiler_params=pltpu.CompilerParams(dimension_semantics=("arbitrary",),
                                             vmem_limit_bytes=64 * 1024 * 1024),
    )(hf0, hb0, hf0, hb0, wcf, bf, wcb, bb, fwf, fwb, fcb)


def _prep_w(wih, whh, bih, bhh):
    wc = jnp.concatenate([wih.T, whh.T], axis=0)
    b = (bih + bhh).reshape(1, 4 * HID)
    scale = jnp.concatenate([jnp.full((HID,), 0.5), jnp.full((HID,), 0.5),
                             jnp.ones((HID,)), jnp.full((HID,), 0.5)]).astype(
                                 jnp.float32)
    wc = wc * scale[None, :]
    b = b * scale[None, :]
    return wc.astype(jnp.bfloat16), b.astype(jnp.bfloat16)


def kernel(x, table, Wih0f, Whh0f, bih0f, bhh0f, Wih0b, Whh0b, bih0b, bhh0b,
           Wih1f, Whh1f, bih1f, bhh1f, Wih1b, Whh1b, bih1b, bhh1b, fcW, fcb):
    idx = x.astype(jnp.int32).T.reshape(1, _N_IDX)
    emb = _sc_gather(table, idx).reshape(T, B, EMB)

    wc0f, b0f = _prep_w(Wih0f, Whh0f, bih0f, bhh0f)
    wc0b, b0b = _prep_w(Wih0b, Whh0b, bih0b, bhh0b)
    hf0, hb0 = _bilstm_layer0(emb, wc0f, b0f, wc0b, b0b)

    wc1f, b1f = _prep_w(Wih1f, Whh1f, bih1f, bhh1f)
    wc1b, b1b = _prep_w(Wih1b, Whh1b, bih1b, bhh1b)

    fwf = fcW[:, :HID]
    fwb = fcW[:, HID:]
    fcbr = fcb.reshape(1, 1)

    return _bilstm_layer1_fc(hf0, hb0, wc1f, b1f, wc1b, b1b, fwf, fwb, fcbr)


# speedup vs baseline: 1.0744x; 1.0744x over previous
import jax
import jax.numpy as jnp
from jax.experimental import pallas as pl
from jax.experimental.pallas import tpu as pltpu
from jax.experimental.pallas import tpu_sc as plsc

EMB = 128
HID = 512
B = 1024
T = 200

_GATHER_WINDOW = 128
_N_IDX = B * T


def _sc_gather(table, idx):
    n = _N_IDX
    e = table.shape[1]
    mesh = plsc.VectorSubcoreMesh(core_axis_name="core", subcore_axis_name="subcore")

    @pl.kernel(out_type=jax.ShapeDtypeStruct((n, e), table.dtype), mesh=mesh)
    def gather_kernel(tab_hbm, i_hbm, o_hbm):
        def body(i_vmem, o_vmem):
            pltpu.sync_copy(tab_hbm.at[i_vmem.at[0]], o_vmem)

        pltpu.emit_pipeline(
            body,
            grid=(n // _GATHER_WINDOW,),
            in_specs=[pl.BlockSpec((1, _GATHER_WINDOW), index_map=lambda i: (0, i))],
            out_specs=[pl.BlockSpec((_GATHER_WINDOW, e), index_map=lambda i: (i, 0))],
            core_axis_name=("core", "subcore"),
            dimension_semantics=(pltpu.PARALLEL,),
        )(i_hbm, o_hbm)

    return gather_kernel(table, idx)


def _gates(g, first, c_ref):
    half = jnp.bfloat16(0.5)
    i = half + half * jnp.tanh(g[:, :HID])
    f = half + half * jnp.tanh(g[:, HID:2 * HID])
    gg = jnp.tanh(g[:, 2 * HID:3 * HID])
    o = half + half * jnp.tanh(g[:, 3 * HID:])
    c_prev = jnp.where(first, jnp.float32(0), c_ref[...])
    c2 = f.astype(jnp.float32) * c_prev + (i * gg).astype(jnp.float32)
    c_ref[...] = c2
    return o * jnp.tanh(c2).astype(jnp.bfloat16)


def _full_spec(a):
    nd = a.ndim
    return pl.BlockSpec(a.shape, lambda t, _n=nd: (0,) * _n)


def _l0_body(xf_ref, xb_ref, wf_ref, bf_ref, wb_ref, bb_ref,
             hfo_ref, hbo_ref, xcf, cf, xcb, cb):
    t = pl.program_id(0)
    first = t == 0

    def step(x_ref, w_ref, b_ref, xc, c, out_ref, j, fst):
        if fst is not False:
            @pl.when(fst)
            def _():
                xc[:, EMB:] = jnp.zeros_like(xc[:, EMB:])

        xc[:, :EMB] = x_ref[j].astype(jnp.bfloat16)
        g = (jnp.dot(xc[...], w_ref[...], preferred_element_type=jnp.float32)
             ).astype(jnp.bfloat16) + b_ref[...]
        hb = _gates(g, fst, c)
        xc[:, EMB:] = hb
        out_ref[j] = hb

    step(xf_ref, wf_ref, bf_ref, xcf, cf, hfo_ref, 0, first)
    step(xb_ref, wb_ref, bb_ref, xcb, cb, hbo_ref, 1, first)
    step(xf_ref, wf_ref, bf_ref, xcf, cf, hfo_ref, 1, False)
    step(xb_ref, wb_ref, bb_ref, xcb, cb, hbo_ref, 0, False)


def _bilstm_layer0(emb, wcf, bf, wcb, bb):
    out_shape = [jax.ShapeDtypeStruct((T, B, HID), jnp.bfloat16),
                 jax.ShapeDtypeStruct((T, B, HID), jnp.bfloat16)]
    k0 = EMB + HID
    return pl.pallas_call(
        _l0_body,
        grid=(T // 2,),
        in_specs=[
            pl.BlockSpec((2, B, EMB), lambda t: (t, 0, 0)),
            pl.BlockSpec((2, B, EMB), lambda t: (T // 2 - 1 - t, 0, 0)),
            _full_spec(wcf), _full_spec(bf), _full_spec(wcb), _full_spec(bb),
        ],
        out_specs=[
            pl.BlockSpec((2, B, HID), lambda t: (t, 0, 0)),
            pl.BlockSpec((2, B, HID), lambda t: (T // 2 - 1 - t, 0, 0)),
        ],
        out_shape=out_shape,
        scratch_shapes=[pltpu.VMEM((B, k0), jnp.bfloat16),
                        pltpu.VMEM((B, HID), jnp.float32),
                        pltpu.VMEM((B, k0), jnp.bfloat16),
                        pltpu.VMEM((B, HID), jnp.float32)],
        compiler_params=pltpu.CompilerParams(dimension_semantics=("arbitrary",)),
    )(emb, emb, wcf, bf, wcb, bb)


def _l1_body(hff_ref, hbf_ref, hfb_ref, hbb_ref,
             wf_ref, bf_ref, wb_ref, bb_ref,
             fwf_ref, fwb_ref, fcb_ref,
             out_ref, xcf, cf, xcb, cb):
    t = pl.program_id(0)
    first = t == 0

    def step(in1_ref, in2_ref, w_ref, b_ref, xc, c, j, fst):
        if fst is not False:
            @pl.when(fst)
            def _():
                xc[:, 2 * HID:] = jnp.zeros_like(xc[:, 2 * HID:])

        xc[:, :HID] = in1_ref[j]
        xc[:, HID:2 * HID] = in2_ref[j]
        g = (jnp.dot(xc[...], w_ref[...], preferred_element_type=jnp.float32)
             ).astype(jnp.bfloat16) + b_ref[...]
        hb = _gates(g, fst, c)
        xc[:, 2 * HID:] = hb

    step(hff_ref, hbf_ref, wf_ref, bf_ref, xcf, cf, 0, first)
    step(hfb_ref, hbb_ref, wb_ref, bb_ref, xcb, cb, 1, first)
    step(hff_ref, hbf_ref, wf_ref, bf_ref, xcf, cf, 1, False)
    step(hfb_ref, hbb_ref, wb_ref, bb_ref, xcb, cb, 0, False)

    @pl.when(t == T // 2 - 1)
    def _():
        vf = jnp.sum(xcf[:, 2 * HID:].astype(jnp.float32) * fwf_ref[...],
                     axis=1, keepdims=True)
        vb = jnp.sum(xcb[:, 2 * HID:].astype(jnp.float32) * fwb_ref[...],
                     axis=1, keepdims=True)
        z = vf + vb + fcb_ref[...]
        out_ref[...] = 0.5 + 0.5 * jnp.tanh(0.5 * z)


def _bilstm_layer1_fc(hf0, hb0, wcf, bf, wcb, bb, fwf, fwb, fcb):
    seq_spec_f = pl.BlockSpec((2, B, HID), lambda t: (t, 0, 0))
    seq_spec_b = pl.BlockSpec((2, B, HID), lambda t: (T // 2 - 1 - t, 0, 0))
    k1 = 3 * HID
    return pl.pallas_call(
        _l1_body,
        grid=(T // 2,),
        in_specs=[
            seq_spec_f, seq_spec_f, seq_spec_b, seq_spec_b,
            _full_spec(wcf), _full_spec(bf), _full_spec(wcb), _full_spec(bb),
            _full_spec(fwf), _full_spec(fwb), _full_spec(fcb),
        ],
        out_specs=pl.BlockSpec((B, 1), lambda t: (0, 0)),
        out_shape=jax.ShapeDtypeStruct((B, 1), jnp.float32),
        scratch_shapes=[pltpu.VMEM((B, k1), jnp.bfloat16),
                        pltpu.VMEM((B, HID), jnp.float32),
                        pltpu.VMEM((B, k1), jnp.bfloat16),
                        pltpu.VMEM((B, HID), jnp.float32)],
        compiler_params=pltpu.CompilerParams(dimension_semantics=("arbitrary",)),
    )(hf0, hb0, hf0, hb0, wcf, bf, wcb, bb, fwf, fwb, fcb)


def _prep_w(wih, whh, bih, bhh):
    wc = jnp.concatenate([wih.T, whh.T], axis=0)
    b = (bih + bhh).reshape(1, 4 * HID)
    scale = jnp.concatenate([jnp.full((HID,), 0.5), jnp.full((HID,), 0.5),
                             jnp.ones((HID,)), jnp.full((HID,), 0.5)]).astype(
                                 jnp.float32)
    wc = wc * scale[None, :]
    b = b * scale[None, :]
    return wc.astype(jnp.bfloat16), b.astype(jnp.bfloat16)


def kernel(x, table, Wih0f, Whh0f, bih0f, bhh0f, Wih0b, Whh0b, bih0b, bhh0b,
           Wih1f, Whh1f, bih1f, bhh1f, Wih1b, Whh1b, bih1b, bhh1b, fcW, fcb):
    idx = x.astype(jnp.int32).T.reshape(1, _N_IDX)
    emb = _sc_gather(table, idx).reshape(T, B, EMB)

    wc0f, b0f = _prep_w(Wih0f, Whh0f, bih0f, bhh0f)
    wc0b, b0b = _prep_w(Wih0b, Whh0b, bih0b, bhh0b)
    hf0, hb0 = _bilstm_layer0(emb, wc0f, b0f, wc0b, b0b)

    wc1f, b1f = _prep_w(Wih1f, Whh1f, bih1f, bhh1f)
    wc1b, b1b = _prep_w(Wih1b, Whh1b, bih1b, bhh1b)

    fwf = fcW[:, :HID]
    fwb = fcW[:, HID:]
    fcbr = fcb.reshape(1, 1)

    return _bilstm_layer1_fc(hf0, hb0, wc1f, b1f, wc1b, b1b, fwf, fwb, fcbr)
